# Initial kernel scaffold; baseline (speedup 1.0000x reference)
import jax
import jax.numpy as jnp
from jax.experimental import pallas as pl


def kernel(x):
    raise NotImplementedError("write your pallas kernel here")


# speedup vs baseline: 2.6704x; 2.6704x over previous
import functools

import jax
import jax.numpy as jnp
from jax import lax
from jax.experimental import pallas as pl


_N = 8192
_LOG2N = 13


def _shift_up(x, d):
    return jnp.concatenate([x[d:], x[:d]], axis=0)


def _shift_down(x, d):
    return jnp.concatenate([x[-d:], x[:-d]], axis=0)


def _bitonic_sort_cols(x):
    n, l = x.shape
    log2n = n.bit_length() - 1
    iota = lax.broadcasted_iota(jnp.int32, (n, 1), 0)
    for k in range(1, log2n + 1):
        for j in range(k - 1, -1, -1):
            d = 1 << j
            if d >= 8:
                m = n // (2 * d)
                x4 = x.reshape(m, 2, d, l)
                a = x4[:, 0]
                b = x4[:, 1]
                mn = jnp.minimum(a, b)
                mx = jnp.maximum(a, b)
                if k == log2n:
                    first, second = mn, mx
                else:
                    mio = lax.broadcasted_iota(jnp.int32, (m, 1, 1), 0)
                    desc = ((mio >> (k - j - 1)) & 1) == 1
                    first = jnp.where(desc, mx, mn)
                    second = jnp.where(desc, mn, mx)
                x = jnp.concatenate(
                    [first[:, None], second[:, None]], axis=1
                ).reshape(n, l)
            else:
                hi_bit = (iota & d) != 0
                if k == log2n:
                    take_max = hi_bit
                else:
                    desc = ((iota >> k) & 1) == 1
                    take_max = hi_bit != desc
                p = jnp.where(hi_bit, _shift_down(x, d), _shift_up(x, d))
                x = jnp.where(take_max, jnp.maximum(x, p), jnp.minimum(x, p))
    return x


def _sort_kernel(x_ref, o_ref):
    o_ref[0] = _bitonic_sort_cols(x_ref[0])


@jax.jit
def kernel(x):
    b, n, f = x.shape
    lblk = 128
    return pl.pallas_call(
        _sort_kernel,
        grid=(b, f // lblk),
        in_specs=[
            pl.BlockSpec((1, n, lblk), lambda i, j: (i, 0, j)),
        ],
        out_specs=pl.BlockSpec((1, n, lblk), lambda i, j: (i, 0, j)),
        out_shape=jax.ShapeDtypeStruct((b, n, f), x.dtype),
    )(x)


# speedup vs baseline: 4.4326x; 1.6599x over previous
import functools

import jax
import jax.numpy as jnp
from jax import lax
from jax.experimental import pallas as pl


_ROT = 3


def _shift_up(x, d):
    return jnp.concatenate([x[d:], x[:d]], axis=0)


def _shift_down(x, d):
    return jnp.concatenate([x[-d:], x[:-d]], axis=0)


def _phys_bit(j, log2n):
    return (j + _ROT) % log2n


def _flip_mask(iota, k, k_next, log2n):
    b = jnp.zeros_like(iota)
    if k < log2n:
        b = b ^ (iota >> _phys_bit(k, log2n))
    if k_next < log2n:
        b = b ^ (iota >> _phys_bit(k_next, log2n))
    return (b & 1) == 1


def _bitonic_sort_cols(x):
    n, l = x.shape
    log2n = n.bit_length() - 1
    iota = lax.broadcasted_iota(jnp.int32, (n, 1), 0)

    m0 = _flip_mask(iota, log2n, 1, log2n)
    x = jnp.where(m0, -x, x)
    for k in range(1, log2n + 1):
        for j in range(k - 1, -1, -1):
            d = 1 << _phys_bit(j, log2n)
            if d >= 8:
                m = n // (2 * d)
                x4 = x.reshape(m, 2, d, l)
                mn = jnp.minimum(x4[:, 0], x4[:, 1])
                mx = jnp.maximum(x4[:, 0], x4[:, 1])
                x = jnp.concatenate(
                    [mn[:, None], mx[:, None]], axis=1
                ).reshape(n, l)
            else:
                hi_bit = (iota & d) != 0
                p = jnp.where(hi_bit, _shift_down(x, d), _shift_up(x, d))
                x = jnp.where(hi_bit, jnp.maximum(x, p), jnp.minimum(x, p))
        mk = _flip_mask(iota, k, k + 1, log2n)
        x = jnp.where(mk, -x, x)
    return x


def _unpermute(x):
    n, l = x.shape
    r = 1 << _ROT
    x3 = x.reshape(n // r, r, l)
    parts = [x3[:, v, :] for v in range(r)]
    return jnp.concatenate(parts, axis=0)


def _sort_kernel(x_ref, o_ref):
    o_ref[0] = _unpermute(_bitonic_sort_cols(x_ref[0]))


@jax.jit
def kernel(x):
    b, n, f = x.shape
    lblk = 128
    return pl.pallas_call(
        _sort_kernel,
        grid=(b, f // lblk),
        in_specs=[
            pl.BlockSpec((1, n, lblk), lambda i, j: (i, 0, j)),
        ],
        out_specs=pl.BlockSpec((1, n, lblk), lambda i, j: (i, 0, j)),
        out_shape=jax.ShapeDtypeStruct((b, n, f), x.dtype),
    )(x)


# speedup vs baseline: 4.6087x; 1.0397x over previous
import functools

import jax
import jax.numpy as jnp
from jax import lax
from jax.experimental import pallas as pl


_ROT = 3


def _shift_up(x, d):
    return jnp.concatenate([x[d:], x[:d]], axis=0)


def _shift_down(x, d):
    return jnp.concatenate([x[-d:], x[:-d]], axis=0)


def _phys_bit(j, log2n):
    return (j + _ROT) % log2n


def _flip_mask(iota, k, k_next, log2n):
    b = jnp.zeros_like(iota)
    if k < log2n:
        b = b ^ (iota >> _phys_bit(k, log2n))
    if k_next < log2n:
        b = b ^ (iota >> _phys_bit(k_next, log2n))
    return (b & 1) == 1


def _cex_group(x, strides):
    n, l = x.shape
    g = len(strides)
    d = strides[-1]
    m = n // (d << g)
    z = x.reshape(m, *([2] * g), d, l)
    pieces = []
    for t in range(1 << g):
        idx = (slice(None),) + tuple((t >> (g - 1 - a)) & 1 for a in range(g))
        pieces.append(z[idx])
    for level in range(g):
        bit = g - 1 - level
        mask = 1 << bit
        for t in range(1 << g):
            if t & mask:
                continue
            a, b = pieces[t], pieces[t | mask]
            pieces[t] = jnp.minimum(a, b)
            pieces[t | mask] = jnp.maximum(a, b)
    return jnp.stack(pieces, axis=1).reshape(n, l)


_GROUP = 4


def _bitonic_sort_cols(x):
    n, l = x.shape
    log2n = n.bit_length() - 1
    iota = lax.broadcasted_iota(jnp.int32, (n, 1), 0)

    m0 = _flip_mask(iota, log2n, 1, log2n)
    x = jnp.where(m0, -x, x)
    for k in range(1, log2n + 1):
        ds = [1 << _phys_bit(j, log2n) for j in range(k - 1, -1, -1)]
        run = []
        for d in ds:
            if d >= 8 and (not run or run[-1] == 2 * d) and len(run) < _GROUP:
                run.append(d)
                continue
            if run:
                x = _cex_group(x, run)
                run = []
            if d >= 8:
                run = [d]
            else:
                hi_bit = (iota & d) != 0
                p = jnp.where(hi_bit, _shift_down(x, d), _shift_up(x, d))
                x = jnp.where(hi_bit, jnp.maximum(x, p), jnp.minimum(x, p))
        if run:
            x = _cex_group(x, run)
        mk = _flip_mask(iota, k, k + 1, log2n)
        x = jnp.where(mk, -x, x)
    return x


def _unpermute(x):
    n, l = x.shape
    r = 1 << _ROT
    x3 = x.reshape(n // r, r, l)
    parts = [x3[:, v, :] for v in range(r)]
    return jnp.concatenate(parts, axis=0)


def _sort_kernel(x_ref, o_ref):
    o_ref[0] = _unpermute(_bitonic_sort_cols(x_ref[0]))


@jax.jit
def kernel(x):
    b, n, f = x.shape
    lblk = 128
    return pl.pallas_call(
        _sort_kernel,
        grid=(b, f // lblk),
        in_specs=[
            pl.BlockSpec((1, n, lblk), lambda i, j: (i, 0, j)),
        ],
        out_specs=pl.BlockSpec((1, n, lblk), lambda i, j: (i, 0, j)),
        out_shape=jax.ShapeDtypeStruct((b, n, f), x.dtype),
    )(x)


# speedup vs baseline: 4.8242x; 1.0468x over previous
import functools

import jax
import jax.numpy as jnp
from jax import lax
from jax.experimental import pallas as pl


_ROT = 3
_GROUP = 4


def _shift_up(x, d):
    return jnp.concatenate([x[d:], x[:d]], axis=0)


def _shift_down(x, d):
    return jnp.concatenate([x[-d:], x[:-d]], axis=0)


def _phys_bit(j, log2n):
    return (j + _ROT) % log2n


def _flip_mask(iota, k, k_next, log2n):
    b = jnp.zeros_like(iota)
    if k < log2n:
        b = b ^ (iota >> _phys_bit(k, log2n))
    if k_next < log2n:
        b = b ^ (iota >> _phys_bit(k_next, log2n))
    return (b & 1) == 1


def _stage_plan(log2n):
    plan = []
    for k in range(1, log2n + 1):
        ds = [1 << _phys_bit(j, log2n) for j in range(k - 1, -1, -1)]
        items = []
        run = []
        for d in ds:
            if d >= 8 and (not run or run[-1] == 2 * d) and len(run) < _GROUP:
                run.append(d)
                continue
            if run:
                items.append(("group", run))
                run = []
            if d >= 8:
                run = [d]
            else:
                items.append(("shift", d))
        if run:
            items.append(("group", run))
        plan.append(items)
    return plan


def _cex_tree(vals):
    g = len(vals).bit_length() - 1
    for level in range(g):
        mask = 1 << (g - 1 - level)
        for t in range(len(vals)):
            if t & mask:
                continue
            a, b = vals[t], vals[t | mask]
            vals[t] = jnp.minimum(a, b)
            vals[t | mask] = jnp.maximum(a, b)
    return vals


def _group_inplace(o_ref, strides, n):
    g = len(strides)
    d = strides[-1]
    npieces = 1 << g
    chunks = d // 8
    iters = n // (8 * npieces)

    def body(i, carry):
        mm = i // chunks
        c = i - mm * chunks
        base = mm * (npieces * d) + c * 8
        vals = [o_ref[0, pl.ds(base + t * d, 8), :] for t in range(npieces)]
        vals = _cex_tree(vals)
        for t in range(npieces):
            o_ref[0, pl.ds(base + t * d, 8), :] = vals[t]
        return carry

    lax.fori_loop(0, iters, body, 0, unroll=False)


def _sort_kernel(x_ref, o_ref):
    n, l = x_ref.shape[1], x_ref.shape[2]
    log2n = n.bit_length() - 1
    iota = lax.broadcasted_iota(jnp.int32, (n, 1), 0)

    x = x_ref[0]
    m0 = _flip_mask(iota, log2n, 1, log2n)
    o_ref[0] = jnp.where(m0, -x, x)

    for k, items in enumerate(_stage_plan(log2n), start=1):
        for kind, arg in items:
            if kind == "group":
                _group_inplace(o_ref, arg, n)
            else:
                d = arg
                x = o_ref[0]
                hi_bit = (iota & d) != 0
                p = jnp.where(hi_bit, _shift_down(x, d), _shift_up(x, d))
                o_ref[0] = jnp.where(
                    hi_bit, jnp.maximum(x, p), jnp.minimum(x, p)
                )
        mk = _flip_mask(iota, k, k + 1, log2n)
        x = o_ref[0]
        o_ref[0] = jnp.where(mk, -x, x)

    x = o_ref[0]
    r = 1 << _ROT
    x3 = x.reshape(n // r, r, l)
    parts = [x3[:, v, :] for v in range(r)]
    o_ref[0] = jnp.concatenate(parts, axis=0)


@jax.jit
def kernel(x):
    b, n, f = x.shape
    lblk = 128
    return pl.pallas_call(
        _sort_kernel,
        grid=(b, f // lblk),
        in_specs=[
            pl.BlockSpec((1, n, lblk), lambda i, j: (i, 0, j)),
        ],
        out_specs=pl.BlockSpec((1, n, lblk), lambda i, j: (i, 0, j)),
        out_shape=jax.ShapeDtypeStruct((b, n, f), x.dtype),
    )(x)


# speedup vs baseline: 6.2983x; 1.3056x over previous
import functools

import jax
import jax.numpy as jnp
from jax import lax
from jax.experimental import pallas as pl


_ROT = 3
_GROUP = 4


def _shift_up(x, d):
    return jnp.concatenate([x[d:], x[:d]], axis=0)


def _shift_down(x, d):
    return jnp.concatenate([x[-d:], x[:-d]], axis=0)


def _phys_bit(j, log2n):
    return (j + _ROT) % log2n


def _flip_mask(iota, k, k_next, log2n):
    b = jnp.zeros_like(iota)
    if k < log2n:
        b = b ^ (iota >> _phys_bit(k, log2n))
    if k_next < log2n:
        b = b ^ (iota >> _phys_bit(k_next, log2n))
    return (b & 1) == 1


def _stage_plan(log2n):
    plan = []
    for k in range(1, log2n + 1):
        ds = [1 << _phys_bit(j, log2n) for j in range(k - 1, -1, -1)]
        items = []
        run = []
        for d in ds:
            if d >= 8 and (not run or run[-1] == 2 * d) and len(run) < _GROUP:
                run.append(d)
                continue
            if run:
                items.append(("group", run))
                run = []
            if d >= 8:
                run = [d]
            else:
                items.append(("shift", d))
        if run:
            items.append(("group", run))
        plan.append(items)
    return plan


def _cex_tree(vals):
    g = len(vals).bit_length() - 1
    for level in range(g):
        mask = 1 << (g - 1 - level)
        for t in range(len(vals)):
            if t & mask:
                continue
            a, b = vals[t], vals[t | mask]
            vals[t] = jnp.minimum(a, b)
            vals[t | mask] = jnp.maximum(a, b)
    return vals


def _group_pass(src, dst, strides, n):
    g = len(strides)
    d = strides[-1]
    npieces = 1 << g
    chunks = d // 8
    iters = n // (8 * npieces)

    def body(i, carry):
        mm = i // chunks
        c = i - mm * chunks
        base = mm * (npieces * d) + c * 8
        vals = [src[pl.ds(base + t * d, 8), :] for t in range(npieces)]
        vals = _cex_tree(vals)
        for t in range(npieces):
            dst[pl.ds(base + t * d, 8), :] = vals[t]
        return carry

    lax.fori_loop(0, iters, body, 0, unroll=2)


def _sort_kernel(x_ref, o_ref, scratch):
    n, l = x_ref.shape[1], x_ref.shape[2]
    log2n = n.bit_length() - 1
    iota = lax.broadcasted_iota(jnp.int32, (n, 1), 0)

    plan = _stage_plan(log2n)

    n_steps = 1 + sum(len(items) for items in plan) + len(plan) + 1
    bufs = [o_ref.at[0], scratch.at[0]] if n_steps % 2 == 0 else [
        scratch.at[0], o_ref.at[0]]
    cur = 0

    def step():
        nonlocal cur
        src, dst = bufs[cur], bufs[1 - cur]
        cur = 1 - cur
        return src, dst

    _, dst = step()
    x = x_ref[0]
    m0 = _flip_mask(iota, log2n, 1, log2n)
    dst[...] = jnp.where(m0, -x, x)

    for k, items in enumerate(plan, start=1):
        for kind, arg in items:
            src, dst = step()
            if kind == "group":
                _group_pass(src, dst, arg, n)
            else:
                d = arg
                x = src[...]
                hi_bit = (iota & d) != 0
                p = jnp.where(hi_bit, _shift_down(x, d), _shift_up(x, d))
                dst[...] = jnp.where(
                    hi_bit, jnp.maximum(x, p), jnp.minimum(x, p)
                )
        src, dst = step()
        mk = _flip_mask(iota, k, k + 1, log2n)
        x = src[...]
        dst[...] = jnp.where(mk, -x, x)

    src, dst = step()
    x = src[...]
    r = 1 << _ROT
    x3 = x.reshape(n // r, r, l)
    parts = [x3[:, v, :] for v in range(r)]
    dst[...] = jnp.concatenate(parts, axis=0)


@jax.jit
def kernel(x):
    b, n, f = x.shape
    lblk = 128
    from jax.experimental.pallas import tpu as pltpu

    return pl.pallas_call(
        _sort_kernel,
        grid=(b, f // lblk),
        in_specs=[
            pl.BlockSpec((1, n, lblk), lambda i, j: (i, 0, j)),
        ],
        out_specs=pl.BlockSpec((1, n, lblk), lambda i, j: (i, 0, j)),
        out_shape=jax.ShapeDtypeStruct((b, n, f), x.dtype),
        scratch_shapes=[pltpu.VMEM((1, n, lblk), x.dtype)],
    )(x)
